# Initial kernel scaffold; baseline (speedup 1.0000x reference)
import jax
import jax.numpy as jnp
from jax.experimental import pallas as pl


def kernel(x, edge_index, edge_weight, idx):
    raise NotImplementedError("write your pallas kernel here")


# speedup vs baseline: 3.3178x; 3.3178x over previous
import functools

import jax
import jax.numpy as jnp
from jax import lax
from jax.experimental import pallas as pl
from jax.experimental.pallas import tpu as pltpu
from jax.experimental.pallas import tpu_sc as plsc

_NC = 2
_NS = 16
_NW = _NC * _NS
_C = 128
_LANES = 16


def _sc_partials(x, row, col, w, n_chunks, e_per_w, n_pad):
    n, d = x.shape
    rows_per_tile = n_pad // _NS
    n_pairs = n_chunks // 2
    mesh = plsc.VectorSubcoreMesh(core_axis_name="c", subcore_axis_name="s",
                                  num_cores=_NC, num_subcores=_NS)

    def body(x_hbm, row_hbm, col_hbm, w_hbm, out_hbm,
             acc, col0, col1, row0, row1, w0, w1, g0, g1, sem0, sem1):
        cid = lax.axis_index("c")
        sid = lax.axis_index("s")
        wid = sid * _NC + cid
        ebase = wid * e_per_w

        zero = jnp.zeros((_LANES,), jnp.float32)

        @pl.loop(0, _C)
        def _zero_buf(i):
            for j in range(d // _LANES):
                g0[i, pl.ds(j * _LANES, _LANES)] = zero

        rbase = sid * rows_per_tile
        nfull = rows_per_tile // _C
        rem = rows_per_tile - nfull * _C
        for k in range(nfull):
            pltpu.sync_copy(g0, acc.at[pl.ds(rbase + k * _C, _C)])
        if rem:
            pltpu.sync_copy(g0.at[pl.ds(0, rem)],
                            acc.at[pl.ds(rbase + nfull * _C, rem)])
        plsc.subcore_barrier()

        def load_issue(ci, colb, rowb, wb, gb, sem):
            eb = ebase + ci * _C
            pltpu.sync_copy(row_hbm.at[pl.ds(eb, _C)], rowb)
            pltpu.sync_copy(col_hbm.at[pl.ds(eb, _C)], colb)
            pltpu.sync_copy(w_hbm.at[pl.ds(eb, _C)], wb)
            pltpu.async_copy(x_hbm.at[colb], gb, sem)

        def process(colb, rowb, wb, gb, sem):
            pltpu.make_async_copy(x_hbm.at[colb], gb, sem).wait()

            @pl.loop(0, _C // _LANES)
            def _scale(g):
                wvec = wb[pl.ds(g * _LANES, _LANES)]
                for k in range(_LANES):
                    e = g * _LANES + k
                    wv = wvec[k]
                    for j in range(d // _LANES):
                        sl = pl.ds(j * _LANES, _LANES)
                        gb[e, sl] = gb[e, sl] * wv

            pltpu.sync_copy(gb, acc.at[rowb], add=True)

        load_issue(0, col0, row0, w0, g0, sem0)

        @pl.loop(0, n_pairs)
        def _main(p):
            load_issue(2 * p + 1, col1, row1, w1, g1, sem1)
            process(col0, row0, w0, g0, sem0)

            @pl.when(p < n_pairs - 1)
            def _():
                load_issue(2 * p + 2, col0, row0, w0, g0, sem0)

            process(col1, row1, w1, g1, sem1)

        plsc.subcore_barrier()
        pltpu.sync_copy(acc.at[pl.ds(rbase, rows_per_tile)],
                        out_hbm.at[cid, pl.ds(rbase, rows_per_tile)])

    kern = pl.kernel(
        body,
        out_type=jax.ShapeDtypeStruct((_NC, n_pad, d), jnp.float32),
        mesh=mesh,
        scratch_types=[
            pltpu.VMEM_SHARED((n_pad, d), jnp.float32),
            pltpu.VMEM((_C,), jnp.int32), pltpu.VMEM((_C,), jnp.int32),
            pltpu.VMEM((_C,), jnp.int32), pltpu.VMEM((_C,), jnp.int32),
            pltpu.VMEM((_C,), jnp.float32), pltpu.VMEM((_C,), jnp.float32),
            pltpu.VMEM((_C, d), jnp.float32), pltpu.VMEM((_C, d), jnp.float32),
            pltpu.SemaphoreType.DMA, pltpu.SemaphoreType.DMA,
        ],
    )
    return kern(x, row, col, w)


def _tc_reduce(partials):
    _, n, d = partials.shape
    br = n // _NS

    def add_body(p_ref, o_ref):
        o_ref[...] = p_ref[0] + p_ref[1]

    return pl.pallas_call(
        add_body,
        grid=(n // br,),
        in_specs=[pl.BlockSpec((_NC, br, d), lambda i: (0, i, 0))],
        out_specs=pl.BlockSpec((br, d), lambda i: (i, 0)),
        out_shape=jax.ShapeDtypeStruct((n, d), jnp.float32),
    )(partials)


def kernel(x, edge_index, edge_weight, idx):
    del idx
    e = edge_weight.shape[0]
    e_per_w = -(-e // (_NW * 2 * _C)) * (2 * _C)
    e_pad = _NW * e_per_w
    pad = e_pad - e
    row = edge_index[0]
    col = edge_index[1]
    if pad:
        zi = jnp.zeros((pad,), row.dtype)
        row = jnp.concatenate([row, zi])
        col = jnp.concatenate([col, zi])
        edge_weight = jnp.concatenate(
            [edge_weight, jnp.zeros((pad,), edge_weight.dtype)])
    n = x.shape[0]
    n_pad = -(-n // (_NS * 8)) * (_NS * 8)
    partials = _sc_partials(x, row, col, edge_weight,
                            e_per_w // _C, e_per_w, n_pad)
    return _tc_reduce(partials)[:n]


# speedup vs baseline: 3.3193x; 1.0004x over previous
import functools

import jax
import jax.numpy as jnp
from jax import lax
from jax.experimental import pallas as pl
from jax.experimental.pallas import tpu as pltpu
from jax.experimental.pallas import tpu_sc as plsc

_NC = 2
_NS = 16
_NW = _NC * _NS
_C = 128
_LANES = 16


def _sc_partials(x, row, col, w, n_chunks, e_per_w, n_pad):
    n, d = x.shape
    rows_per_tile = n_pad // _NS
    n_pairs = n_chunks // 2
    mesh = plsc.VectorSubcoreMesh(core_axis_name="c", subcore_axis_name="s",
                                  num_cores=_NC, num_subcores=_NS)

    def body(x_hbm, row_hbm, col_hbm, w_hbm, out_hbm,
             acc, col0, col1, row0, row1, w0, w1, g0, g1,
             sem0, sem1, ssem0, ssem1):
        cid = lax.axis_index("c")
        sid = lax.axis_index("s")
        wid = sid * _NC + cid
        ebase = wid * e_per_w

        zero = jnp.zeros((_LANES,), jnp.float32)

        @pl.loop(0, _C)
        def _zero_buf(i):
            for j in range(d // _LANES):
                g0[i, pl.ds(j * _LANES, _LANES)] = zero

        rbase = sid * rows_per_tile
        nfull = rows_per_tile // _C
        rem = rows_per_tile - nfull * _C
        for k in range(nfull):
            pltpu.sync_copy(g0, acc.at[pl.ds(rbase + k * _C, _C)])
        if rem:
            pltpu.sync_copy(g0.at[pl.ds(0, rem)],
                            acc.at[pl.ds(rbase + nfull * _C, rem)])
        plsc.subcore_barrier()

        def load_issue(ci, colb, rowb, wb, gb, sem):
            eb = ebase + ci * _C
            pltpu.sync_copy(row_hbm.at[pl.ds(eb, _C)], rowb)
            pltpu.sync_copy(col_hbm.at[pl.ds(eb, _C)], colb)
            pltpu.sync_copy(w_hbm.at[pl.ds(eb, _C)], wb)
            pltpu.async_copy(x_hbm.at[colb], gb, sem)

        def process(colb, rowb, wb, gb, sem, ssem):
            pltpu.make_async_copy(x_hbm.at[colb], gb, sem).wait()

            @pl.loop(0, _C // _LANES)
            def _scale(g):
                wvec = wb[pl.ds(g * _LANES, _LANES)]
                for k in range(_LANES):
                    e = g * _LANES + k
                    wv = wvec[k]
                    for j in range(d // _LANES):
                        sl = pl.ds(j * _LANES, _LANES)
                        gb[e, sl] = gb[e, sl] * wv

            pltpu.async_copy(gb, acc.at[rowb], sem=ssem, add=True)

        def drain(gb, rowb, ssem):
            pltpu.make_async_copy(gb, acc.at[rowb], ssem).wait()

        load_issue(0, col0, row0, w0, g0, sem0)

        @pl.loop(0, n_pairs)
        def _main(p):
            @pl.when(p > 0)
            def _():
                drain(g1, row1, ssem1)

            load_issue(2 * p + 1, col1, row1, w1, g1, sem1)
            process(col0, row0, w0, g0, sem0, ssem0)

            @pl.when(p < n_pairs - 1)
            def _():
                drain(g0, row0, ssem0)
                load_issue(2 * p + 2, col0, row0, w0, g0, sem0)

            process(col1, row1, w1, g1, sem1, ssem1)

        drain(g0, row0, ssem0)
        drain(g1, row1, ssem1)
        plsc.subcore_barrier()
        pltpu.sync_copy(acc.at[pl.ds(rbase, rows_per_tile)],
                        out_hbm.at[cid, pl.ds(rbase, rows_per_tile)])

    kern = pl.kernel(
        body,
        out_type=jax.ShapeDtypeStruct((_NC, n_pad, d), jnp.float32),
        mesh=mesh,
        scratch_types=[
            pltpu.VMEM_SHARED((n_pad, d), jnp.float32),
            pltpu.VMEM((_C,), jnp.int32), pltpu.VMEM((_C,), jnp.int32),
            pltpu.VMEM((_C,), jnp.int32), pltpu.VMEM((_C,), jnp.int32),
            pltpu.VMEM((_C,), jnp.float32), pltpu.VMEM((_C,), jnp.float32),
            pltpu.VMEM((_C, d), jnp.float32), pltpu.VMEM((_C, d), jnp.float32),
            pltpu.SemaphoreType.DMA, pltpu.SemaphoreType.DMA,
            pltpu.SemaphoreType.DMA, pltpu.SemaphoreType.DMA,
        ],
    )
    return kern(x, row, col, w)


def _tc_reduce(partials):
    _, n, d = partials.shape
    br = n // _NS

    def add_body(p_ref, o_ref):
        o_ref[...] = p_ref[0] + p_ref[1]

    return pl.pallas_call(
        add_body,
        grid=(n // br,),
        in_specs=[pl.BlockSpec((_NC, br, d), lambda i: (0, i, 0))],
        out_specs=pl.BlockSpec((br, d), lambda i: (i, 0)),
        out_shape=jax.ShapeDtypeStruct((n, d), jnp.float32),
    )(partials)


def kernel(x, edge_index, edge_weight, idx):
    del idx
    e = edge_weight.shape[0]
    e_per_w = -(-e // (_NW * 2 * _C)) * (2 * _C)
    e_pad = _NW * e_per_w
    pad = e_pad - e
    row = edge_index[0]
    col = edge_index[1]
    if pad:
        zi = jnp.zeros((pad,), row.dtype)
        row = jnp.concatenate([row, zi])
        col = jnp.concatenate([col, zi])
        edge_weight = jnp.concatenate(
            [edge_weight, jnp.zeros((pad,), edge_weight.dtype)])
    n = x.shape[0]
    n_pad = -(-n // (_NS * 8)) * (_NS * 8)
    partials = _sc_partials(x, row, col, edge_weight,
                            e_per_w // _C, e_per_w, n_pad)
    return _tc_reduce(partials)[:n]


# speedup vs baseline: 3.4184x; 1.0299x over previous
import functools

import jax
import jax.numpy as jnp
from jax import lax
from jax.experimental import pallas as pl
from jax.experimental.pallas import tpu as pltpu
from jax.experimental.pallas import tpu_sc as plsc

_NC = 2
_NS = 16
_NW = _NC * _NS
_C = 128
_LANES = 16


def _sc_partials(x, row, col, w, n_chunks, e_per_w, n_pad):
    n, d = x.shape
    rows_per_tile = n_pad // _NS
    n_pairs = n_chunks // 2
    mesh = plsc.VectorSubcoreMesh(core_axis_name="c", subcore_axis_name="s",
                                  num_cores=_NC, num_subcores=_NS)

    def body(x_hbm, row_hbm, col_hbm, w_hbm, out_hbm,
             acc, col0, col1, row0, row1, w0, w1, g0, g1,
             sem0, sem1, ssem0, ssem1, esem):
        cid = lax.axis_index("c")
        sid = lax.axis_index("s")
        wid = sid * _NC + cid
        ebase = wid * e_per_w

        zero = jnp.zeros((_LANES,), jnp.float32)

        @pl.loop(0, _C)
        def _zero_buf(i):
            for j in range(d // _LANES):
                g0[i, pl.ds(j * _LANES, _LANES)] = zero

        rbase = sid * rows_per_tile
        nfull = rows_per_tile // _C
        rem = rows_per_tile - nfull * _C
        for k in range(nfull):
            pltpu.sync_copy(g0, acc.at[pl.ds(rbase + k * _C, _C)])
        if rem:
            pltpu.sync_copy(g0.at[pl.ds(0, rem)],
                            acc.at[pl.ds(rbase + nfull * _C, rem)])
        plsc.subcore_barrier()

        def load_issue(ci, colb, rowb, wb, gb, sem, esem):
            eb = ebase + ci * _C
            pltpu.async_copy(row_hbm.at[pl.ds(eb, _C)], rowb, esem)
            pltpu.async_copy(col_hbm.at[pl.ds(eb, _C)], colb, esem)
            pltpu.async_copy(w_hbm.at[pl.ds(eb, _C)], wb, esem)
            pltpu.make_async_copy(row_hbm.at[pl.ds(eb, _C)], rowb, esem).wait()
            pltpu.make_async_copy(col_hbm.at[pl.ds(eb, _C)], colb, esem).wait()
            pltpu.make_async_copy(w_hbm.at[pl.ds(eb, _C)], wb, esem).wait()
            pltpu.async_copy(x_hbm.at[colb], gb, sem)

        def process(colb, rowb, wb, gb, sem, ssem):
            pltpu.make_async_copy(x_hbm.at[colb], gb, sem).wait()

            @pl.loop(0, _C // _LANES)
            def _scale(g):
                wvec = wb[pl.ds(g * _LANES, _LANES)]
                for k in range(_LANES):
                    e = g * _LANES + k
                    wv = wvec[k]
                    for j in range(d // _LANES):
                        sl = pl.ds(j * _LANES, _LANES)
                        gb[e, sl] = gb[e, sl] * wv

            pltpu.async_copy(gb, acc.at[rowb], sem=ssem, add=True)

        def drain(gb, rowb, ssem):
            pltpu.make_async_copy(gb, acc.at[rowb], ssem).wait()

        load_issue(0, col0, row0, w0, g0, sem0, esem)

        @pl.loop(0, n_pairs)
        def _main(p):
            @pl.when(p > 0)
            def _():
                drain(g1, row1, ssem1)

            load_issue(2 * p + 1, col1, row1, w1, g1, sem1, esem)
            process(col0, row0, w0, g0, sem0, ssem0)

            @pl.when(p < n_pairs - 1)
            def _():
                drain(g0, row0, ssem0)
                load_issue(2 * p + 2, col0, row0, w0, g0, sem0, esem)

            process(col1, row1, w1, g1, sem1, ssem1)

        drain(g0, row0, ssem0)
        drain(g1, row1, ssem1)
        plsc.subcore_barrier()
        pltpu.sync_copy(acc.at[pl.ds(rbase, rows_per_tile)],
                        out_hbm.at[cid, pl.ds(rbase, rows_per_tile)])

    kern = pl.kernel(
        body,
        out_type=jax.ShapeDtypeStruct((_NC, n_pad, d), jnp.float32),
        mesh=mesh,
        scratch_types=[
            pltpu.VMEM_SHARED((n_pad, d), jnp.float32),
            pltpu.VMEM((_C,), jnp.int32), pltpu.VMEM((_C,), jnp.int32),
            pltpu.VMEM((_C,), jnp.int32), pltpu.VMEM((_C,), jnp.int32),
            pltpu.VMEM((_C,), jnp.float32), pltpu.VMEM((_C,), jnp.float32),
            pltpu.VMEM((_C, d), jnp.float32), pltpu.VMEM((_C, d), jnp.float32),
            pltpu.SemaphoreType.DMA, pltpu.SemaphoreType.DMA,
            pltpu.SemaphoreType.DMA, pltpu.SemaphoreType.DMA,
            pltpu.SemaphoreType.DMA,
        ],
    )
    return kern(x, row, col, w)


def _tc_reduce(partials):
    _, n, d = partials.shape
    br = n // _NS

    def add_body(p_ref, o_ref):
        o_ref[...] = p_ref[0] + p_ref[1]

    return pl.pallas_call(
        add_body,
        grid=(n // br,),
        in_specs=[pl.BlockSpec((_NC, br, d), lambda i: (0, i, 0))],
        out_specs=pl.BlockSpec((br, d), lambda i: (i, 0)),
        out_shape=jax.ShapeDtypeStruct((n, d), jnp.float32),
    )(partials)


def kernel(x, edge_index, edge_weight, idx):
    del idx
    e = edge_weight.shape[0]
    e_per_w = -(-e // (_NW * 2 * _C)) * (2 * _C)
    e_pad = _NW * e_per_w
    pad = e_pad - e
    row = edge_index[0]
    col = edge_index[1]
    if pad:
        zi = jnp.zeros((pad,), row.dtype)
        row = jnp.concatenate([row, zi])
        col = jnp.concatenate([col, zi])
        edge_weight = jnp.concatenate(
            [edge_weight, jnp.zeros((pad,), edge_weight.dtype)])
    n = x.shape[0]
    n_pad = -(-n // (_NS * 8)) * (_NS * 8)
    partials = _sc_partials(x, row, col, edge_weight,
                            e_per_w // _C, e_per_w, n_pad)
    return _tc_reduce(partials)[:n]


# speedup vs baseline: 3.4225x; 1.0012x over previous
import functools

import jax
import jax.numpy as jnp
from jax import lax
from jax.experimental import pallas as pl
from jax.experimental.pallas import tpu as pltpu
from jax.experimental.pallas import tpu_sc as plsc

_NC = 2
_NS = 16
_NW = _NC * _NS
_C = 128
_LANES = 16


def _sc_partials(x, row, col, w, n_chunks, e_per_w, n_pad):
    n, d = x.shape
    rows_per_tile = n_pad // _NS
    n_pairs = n_chunks // 2
    mesh = plsc.VectorSubcoreMesh(core_axis_name="c", subcore_axis_name="s",
                                  num_cores=_NC, num_subcores=_NS)

    def body(x_hbm, row_hbm, col_hbm, w_hbm, out_hbm,
             acc, col0, col1, row0, row1, w0, w1, g0, g1,
             sem0, sem1, ssem0, ssem1, esem):
        cid = lax.axis_index("c")
        sid = lax.axis_index("s")
        wid = sid * _NC + cid
        ebase = wid * e_per_w

        zero = jnp.zeros((_LANES,), jnp.float32)

        @pl.loop(0, _C)
        def _zero_buf(i):
            for j in range(d // _LANES):
                g0[i, pl.ds(j * _LANES, _LANES)] = zero

        rbase = sid * rows_per_tile
        nfull = rows_per_tile // _C
        rem = rows_per_tile - nfull * _C
        for k in range(nfull):
            pltpu.sync_copy(g0, acc.at[pl.ds(rbase + k * _C, _C)])
        if rem:
            pltpu.sync_copy(g0.at[pl.ds(0, rem)],
                            acc.at[pl.ds(rbase + nfull * _C, rem)])
        plsc.subcore_barrier()

        def load_issue(ci, colb, rowb, wb, gb, sem, esem):
            eb = ebase + ci * _C
            pltpu.async_copy(row_hbm.at[pl.ds(eb, _C)], rowb, esem)
            pltpu.async_copy(col_hbm.at[pl.ds(eb, _C)], colb, esem)
            pltpu.async_copy(w_hbm.at[pl.ds(eb, _C)], wb, esem)
            pltpu.make_async_copy(row_hbm.at[pl.ds(eb, _C)], rowb, esem).wait()
            pltpu.make_async_copy(col_hbm.at[pl.ds(eb, _C)], colb, esem).wait()
            pltpu.make_async_copy(w_hbm.at[pl.ds(eb, _C)], wb, esem).wait()
            pltpu.async_copy(x_hbm.at[colb], gb, sem)

        def process(colb, rowb, wb, gb, sem, ssem):
            pltpu.make_async_copy(x_hbm.at[colb], gb, sem).wait()

            @plsc.parallel_loop(0, _C // _LANES, unroll=2)
            def _scale(g):
                wvec = wb[pl.ds(g * _LANES, _LANES)]
                for k in range(_LANES):
                    e = g * _LANES + k
                    wv = wvec[k]
                    for j in range(d // _LANES):
                        sl = pl.ds(j * _LANES, _LANES)
                        gb[e, sl] = gb[e, sl] * wv

            pltpu.async_copy(gb, acc.at[rowb], sem=ssem, add=True)

        def drain(gb, rowb, ssem):
            pltpu.make_async_copy(gb, acc.at[rowb], ssem).wait()

        load_issue(0, col0, row0, w0, g0, sem0, esem)

        @pl.loop(0, n_pairs)
        def _main(p):
            @pl.when(p > 0)
            def _():
                drain(g1, row1, ssem1)

            load_issue(2 * p + 1, col1, row1, w1, g1, sem1, esem)
            process(col0, row0, w0, g0, sem0, ssem0)

            @pl.when(p < n_pairs - 1)
            def _():
                drain(g0, row0, ssem0)
                load_issue(2 * p + 2, col0, row0, w0, g0, sem0, esem)

            process(col1, row1, w1, g1, sem1, ssem1)

        drain(g0, row0, ssem0)
        drain(g1, row1, ssem1)
        plsc.subcore_barrier()
        pltpu.sync_copy(acc.at[pl.ds(rbase, rows_per_tile)],
                        out_hbm.at[cid, pl.ds(rbase, rows_per_tile)])

    kern = pl.kernel(
        body,
        out_type=jax.ShapeDtypeStruct((_NC, n_pad, d), jnp.float32),
        mesh=mesh,
        scratch_types=[
            pltpu.VMEM_SHARED((n_pad, d), jnp.float32),
            pltpu.VMEM((_C,), jnp.int32), pltpu.VMEM((_C,), jnp.int32),
            pltpu.VMEM((_C,), jnp.int32), pltpu.VMEM((_C,), jnp.int32),
            pltpu.VMEM((_C,), jnp.float32), pltpu.VMEM((_C,), jnp.float32),
            pltpu.VMEM((_C, d), jnp.float32), pltpu.VMEM((_C, d), jnp.float32),
            pltpu.SemaphoreType.DMA, pltpu.SemaphoreType.DMA,
            pltpu.SemaphoreType.DMA, pltpu.SemaphoreType.DMA,
            pltpu.SemaphoreType.DMA,
        ],
    )
    return kern(x, row, col, w)


def _tc_reduce(partials):
    _, n, d = partials.shape
    br = n // _NS

    def add_body(p_ref, o_ref):
        o_ref[...] = p_ref[0] + p_ref[1]

    return pl.pallas_call(
        add_body,
        grid=(n // br,),
        in_specs=[pl.BlockSpec((_NC, br, d), lambda i: (0, i, 0))],
        out_specs=pl.BlockSpec((br, d), lambda i: (i, 0)),
        out_shape=jax.ShapeDtypeStruct((n, d), jnp.float32),
    )(partials)


def kernel(x, edge_index, edge_weight, idx):
    del idx
    e = edge_weight.shape[0]
    e_per_w = -(-e // (_NW * 2 * _C)) * (2 * _C)
    e_pad = _NW * e_per_w
    pad = e_pad - e
    row = edge_index[0]
    col = edge_index[1]
    if pad:
        zi = jnp.zeros((pad,), row.dtype)
        row = jnp.concatenate([row, zi])
        col = jnp.concatenate([col, zi])
        edge_weight = jnp.concatenate(
            [edge_weight, jnp.zeros((pad,), edge_weight.dtype)])
    n = x.shape[0]
    n_pad = -(-n // (_NS * 8)) * (_NS * 8)
    partials = _sc_partials(x, row, col, edge_weight,
                            e_per_w // _C, e_per_w, n_pad)
    return _tc_reduce(partials)[:n]
